# Initial kernel scaffold; baseline (speedup 1.0000x reference)
import jax
import jax.numpy as jnp
from jax.experimental import pallas as pl


def kernel(entity_ids, table, W, gamma, beta):
    raise NotImplementedError("write your pallas kernel here")


# speedup vs baseline: 1.1947x; 1.1947x over previous
import functools

import jax
import jax.numpy as jnp
from jax import lax
from jax.experimental import pallas as pl
from jax.experimental.pallas import tpu as pltpu
from jax.experimental.pallas import tpu_sc as plsc

_EPS = 1e-12


@functools.lru_cache(maxsize=None)
def _make_sc_gather(N: int, V: int, D: int):
    info = plsc.get_sparse_core_info()
    NC, NS = info.num_cores, info.num_subcores
    NW = NC * NS
    assert N % NW == 0
    b_per_w = N // NW
    CH = 640
    while b_per_w % CH:
        CH //= 2
    n_ch = b_per_w // CH
    mesh = plsc.VectorSubcoreMesh(core_axis_name="c", subcore_axis_name="s")

    @functools.partial(
        pl.kernel,
        mesh=mesh,
        out_type=jax.ShapeDtypeStruct((N, D), jnp.float32),
        scratch_types=[
            pltpu.VMEM((CH,), jnp.int32),
            pltpu.VMEM((CH, D), jnp.float32),
            pltpu.SemaphoreType.DMA,
        ],
    )
    def gather_kernel(idx_hbm, table_hbm, out_hbm, idx_v, rows_v, sem):
        wid = lax.axis_index("s") * NC + lax.axis_index("c")
        base = wid * b_per_w

        def body(i, carry):
            off = base + i * CH
            pltpu.sync_copy(idx_hbm.at[pl.ds(off, CH)], idx_v)
            pltpu.async_copy(table_hbm.at[idx_v], rows_v, sem).wait()
            pltpu.sync_copy(rows_v, out_hbm.at[pl.ds(off, CH)])
            return carry

        lax.fori_loop(0, n_ch, body, 0)

    return gather_kernel


def _proj_ln_body(g_ref, w_ref, gamma_ref, beta_ref, o_ref):
    g = g_ref[...]
    w = w_ref[...]
    h = jnp.dot(g, w, preferred_element_type=jnp.float32)
    mean = jnp.mean(h, axis=-1, keepdims=True)
    c = h - mean
    var = jnp.mean(c * c, axis=-1, keepdims=True)
    o_ref[...] = (c * lax.rsqrt(var + _EPS)) * gamma_ref[...] + beta_ref[...]


def _proj_ln(g, W, gamma, beta, T: int = 256):
    N, D = g.shape
    H = W.shape[1]
    return pl.pallas_call(
        _proj_ln_body,
        grid=(N // T,),
        in_specs=[
            pl.BlockSpec((T, D), lambda i: (i, 0)),
            pl.BlockSpec((D, H), lambda i: (0, 0)),
            pl.BlockSpec((1, H), lambda i: (0, 0)),
            pl.BlockSpec((1, H), lambda i: (0, 0)),
        ],
        out_specs=pl.BlockSpec((T, H), lambda i: (i, 0)),
        out_shape=jax.ShapeDtypeStruct((N, H), jnp.float32),
    )(g, W, gamma.reshape(1, H), beta.reshape(1, H))


def kernel(entity_ids, table, W, gamma, beta):
    B, L = entity_ids.shape
    N = B * L
    V, D = table.shape
    H = W.shape[1]
    idx = entity_ids.reshape(N).astype(jnp.int32)
    g = _make_sc_gather(N, V, D)(idx, table)
    out = _proj_ln(g, W, gamma, beta)
    return out.reshape(B, L, H)


# speedup vs baseline: 2.9452x; 2.4652x over previous
import functools

import jax
import jax.numpy as jnp
from jax import lax
from jax.experimental import pallas as pl
from jax.experimental.pallas import tpu as pltpu
from jax.experimental.pallas import tpu_sc as plsc

_EPS = 1e-12


@functools.lru_cache(maxsize=None)
def _make_sc_gather(N: int, V: int, D: int):
    info = plsc.get_sparse_core_info()
    NC, NS = info.num_cores, info.num_subcores
    NW = NC * NS
    assert N % NW == 0
    b_per_w = N // NW
    CH = 640
    while b_per_w % CH:
        CH //= 2
    n_ch = b_per_w // CH
    mesh = plsc.VectorSubcoreMesh(core_axis_name="c", subcore_axis_name="s")

    @functools.partial(
        pl.kernel,
        mesh=mesh,
        out_type=jax.ShapeDtypeStruct((N, D), jnp.float32),
        scratch_types=[
            pltpu.VMEM((CH,), jnp.int32),
            pltpu.VMEM((CH, D), jnp.float32),
            pltpu.SemaphoreType.DMA,
        ],
    )
    def gather_kernel(idx_hbm, table_hbm, out_hbm, idx_v, rows_v, sem):
        wid = lax.axis_index("s") * NC + lax.axis_index("c")
        base = wid * b_per_w

        def body(i, carry):
            off = base + i * CH
            pltpu.sync_copy(idx_hbm.at[pl.ds(off, CH)], idx_v)
            pltpu.async_copy(table_hbm.at[idx_v], rows_v, sem).wait()
            pltpu.sync_copy(rows_v, out_hbm.at[pl.ds(off, CH)])
            return carry

        lax.fori_loop(0, n_ch, body, 0)

    return gather_kernel


def _proj_ln_body(g_ref, w_ref, gamma_ref, beta_ref, o_ref):
    g = g_ref[...]
    w = w_ref[...]
    h = jnp.dot(g, w, preferred_element_type=jnp.float32)
    mean = jnp.mean(h, axis=-1, keepdims=True)
    c = h - mean
    var = jnp.mean(c * c, axis=-1, keepdims=True)
    o_ref[...] = (c * lax.rsqrt(var + _EPS)) * gamma_ref[...] + beta_ref[...]


def _proj_ln(g, W, gamma, beta, T: int = 256):
    N, D = g.shape
    H = W.shape[1]
    return pl.pallas_call(
        _proj_ln_body,
        grid=(N // T,),
        in_specs=[
            pl.BlockSpec((T, D), lambda i: (i, 0)),
            pl.BlockSpec((D, H), lambda i: (0, 0)),
            pl.BlockSpec((1, H), lambda i: (0, 0)),
            pl.BlockSpec((1, H), lambda i: (0, 0)),
        ],
        out_specs=pl.BlockSpec((T, H), lambda i: (i, 0)),
        out_shape=jax.ShapeDtypeStruct((N, H), jnp.float32),
    )(g, W, gamma.reshape(1, H), beta.reshape(1, H))


def kernel(entity_ids, table, W, gamma, beta):
    B, L = entity_ids.shape
    N = B * L
    V, D = table.shape
    H = W.shape[1]
    idx = entity_ids.T.reshape(N).astype(jnp.int32)
    g = _make_sc_gather(N, V, D)(idx, table)
    out = _proj_ln(g, W, gamma, beta)
    return out.reshape(L, B, H).transpose(1, 0, 2)


# speedup vs baseline: 4.0867x; 1.3876x over previous
import functools

import jax
import jax.numpy as jnp
from jax import lax
from jax.experimental import pallas as pl
from jax.experimental.pallas import tpu as pltpu
from jax.experimental.pallas import tpu_sc as plsc

_EPS = 1e-12


@functools.lru_cache(maxsize=None)
def _make_sc_gather(N: int, V: int, D: int):
    info = plsc.get_sparse_core_info()
    NC, NS = info.num_cores, info.num_subcores
    NW = NC * NS
    assert N % NW == 0
    b_per_w = N // NW
    CH = 640
    while b_per_w % CH:
        CH //= 2
    n_ch = b_per_w // CH
    mesh = plsc.VectorSubcoreMesh(core_axis_name="c", subcore_axis_name="s")

    @functools.partial(
        pl.kernel,
        mesh=mesh,
        out_type=jax.ShapeDtypeStruct((N, D), jnp.float32),
        scratch_types=[
            pltpu.VMEM((CH,), jnp.int32),
            pltpu.VMEM((CH, D), jnp.float32),
            pltpu.SemaphoreType.DMA,
        ],
    )
    def gather_kernel(idx_hbm, table_hbm, out_hbm, idx_v, rows_v, sem):
        wid = lax.axis_index("s") * NC + lax.axis_index("c")
        base = wid * b_per_w

        def body(i, carry):
            off = base + i * CH
            pltpu.sync_copy(idx_hbm.at[pl.ds(off, CH)], idx_v)
            pltpu.async_copy(table_hbm.at[idx_v], rows_v, sem).wait()
            pltpu.sync_copy(rows_v, out_hbm.at[pl.ds(off, CH)])
            return carry

        lax.fori_loop(0, n_ch, body, 0)

    return gather_kernel


def _proj_ln_body(g_ref, w_ref, gamma_ref, beta_ref, o_ref):
    g = g_ref[...]
    w = w_ref[...]
    h = jnp.dot(g, w, preferred_element_type=jnp.float32)
    mean = jnp.mean(h, axis=-1, keepdims=True)
    c = h - mean
    var = jnp.mean(c * c, axis=-1, keepdims=True)
    o_ref[...] = (c * lax.rsqrt(var + _EPS)) * gamma_ref[...] + beta_ref[...]


def _proj_ln(g, W, gamma, beta, T: int = 512):
    N, D = g.shape
    H = W.shape[1]
    return pl.pallas_call(
        _proj_ln_body,
        grid=(N // T,),
        in_specs=[
            pl.BlockSpec((T, D), lambda i: (i, 0)),
            pl.BlockSpec((D, H), lambda i: (0, 0)),
            pl.BlockSpec((1, H), lambda i: (0, 0)),
            pl.BlockSpec((1, H), lambda i: (0, 0)),
        ],
        out_specs=pl.BlockSpec((T, H), lambda i: (i, 0)),
        out_shape=jax.ShapeDtypeStruct((N, H), jnp.float32),
    )(g, W, gamma.reshape(1, H), beta.reshape(1, H))


def kernel(entity_ids, table, W, gamma, beta):
    B, L = entity_ids.shape
    N = B * L
    V, D = table.shape
    H = W.shape[1]
    idx = entity_ids.T.reshape(N).astype(jnp.int32)
    g = _make_sc_gather(N, V, D)(idx, table)
    out = _proj_ln(g, W, gamma, beta)
    return out.reshape(L, B, H).transpose(1, 0, 2)


# speedup vs baseline: 5.0362x; 1.2323x over previous
import functools

import jax
import jax.numpy as jnp
from jax import lax
from jax.experimental import pallas as pl
from jax.experimental.pallas import tpu as pltpu
from jax.experimental.pallas import tpu_sc as plsc

_EPS = 1e-12


@functools.lru_cache(maxsize=None)
def _make_sc_gather(N: int, V: int, D: int):
    info = plsc.get_sparse_core_info()
    NC, NS = info.num_cores, info.num_subcores
    NW = NC * NS
    assert N % NW == 0
    b_per_w = N // NW
    CH = 640
    while b_per_w % CH:
        CH //= 2
    n_ch = b_per_w // CH
    mesh = plsc.VectorSubcoreMesh(core_axis_name="c", subcore_axis_name="s")

    @functools.partial(
        pl.kernel,
        mesh=mesh,
        out_type=jax.ShapeDtypeStruct((N, D), jnp.float32),
        scratch_types=[
            pltpu.VMEM((CH,), jnp.int32),
            pltpu.VMEM((CH, D), jnp.float32),
            pltpu.SemaphoreType.DMA,
        ],
    )
    def gather_kernel(idx_hbm, table_hbm, out_hbm, idx_v, rows_v, sem):
        wid = lax.axis_index("s") * NC + lax.axis_index("c")
        base = wid * b_per_w

        def body(i, carry):
            off = base + i * CH
            pltpu.sync_copy(idx_hbm.at[pl.ds(off, CH)], idx_v)
            pltpu.async_copy(table_hbm.at[idx_v], rows_v, sem).wait()
            pltpu.sync_copy(rows_v, out_hbm.at[pl.ds(off, CH)])
            return carry

        lax.fori_loop(0, n_ch, body, 0)

    return gather_kernel


def _proj_ln_body(g_ref, w_ref, gamma_ref, beta_ref, o_ref):
    g = g_ref[...]
    w = w_ref[...]
    h = jnp.dot(g, w, preferred_element_type=jnp.float32)
    mean = jnp.mean(h, axis=-1, keepdims=True)
    c = h - mean
    var = jnp.mean(c * c, axis=-1, keepdims=True)
    o_ref[...] = (c * lax.rsqrt(var + _EPS)) * gamma_ref[...] + beta_ref[...]


def _proj_ln(g, W, gamma, beta, T: int = 1024):
    N, D = g.shape
    H = W.shape[1]
    return pl.pallas_call(
        _proj_ln_body,
        grid=(N // T,),
        in_specs=[
            pl.BlockSpec((T, D), lambda i: (i, 0)),
            pl.BlockSpec((D, H), lambda i: (0, 0)),
            pl.BlockSpec((1, H), lambda i: (0, 0)),
            pl.BlockSpec((1, H), lambda i: (0, 0)),
        ],
        out_specs=pl.BlockSpec((T, H), lambda i: (i, 0)),
        out_shape=jax.ShapeDtypeStruct((N, H), jnp.float32),
    )(g, W, gamma.reshape(1, H), beta.reshape(1, H))


def kernel(entity_ids, table, W, gamma, beta):
    B, L = entity_ids.shape
    N = B * L
    V, D = table.shape
    H = W.shape[1]
    idx = entity_ids.T.reshape(N).astype(jnp.int32)
    g = _make_sc_gather(N, V, D)(idx, table)
    out = _proj_ln(g, W, gamma, beta)
    return out.reshape(L, B, H).transpose(1, 0, 2)


# speedup vs baseline: 5.7316x; 1.1381x over previous
import functools

import jax
import jax.numpy as jnp
from jax import lax
from jax.experimental import pallas as pl
from jax.experimental.pallas import tpu as pltpu
from jax.experimental.pallas import tpu_sc as plsc

_EPS = 1e-12


@functools.lru_cache(maxsize=None)
def _make_sc_gather(N: int, V: int, D: int):
    info = plsc.get_sparse_core_info()
    NC, NS = info.num_cores, info.num_subcores
    NW = NC * NS
    assert N % NW == 0
    b_per_w = N // NW
    CH = 640
    while b_per_w % CH:
        CH //= 2
    n_ch = b_per_w // CH
    mesh = plsc.VectorSubcoreMesh(core_axis_name="c", subcore_axis_name="s")

    @functools.partial(
        pl.kernel,
        mesh=mesh,
        out_type=jax.ShapeDtypeStruct((N, D), jnp.float32),
        scratch_types=[
            pltpu.VMEM((CH,), jnp.int32),
            pltpu.VMEM((CH, D), jnp.float32),
            pltpu.SemaphoreType.DMA,
        ],
    )
    def gather_kernel(idx_hbm, table_hbm, out_hbm, idx_v, rows_v, sem):
        wid = lax.axis_index("s") * NC + lax.axis_index("c")
        base = wid * b_per_w

        def body(i, carry):
            off = base + i * CH
            pltpu.sync_copy(idx_hbm.at[pl.ds(off, CH)], idx_v)
            pltpu.async_copy(table_hbm.at[idx_v], rows_v, sem).wait()
            pltpu.sync_copy(rows_v, out_hbm.at[pl.ds(off, CH)])
            return carry

        lax.fori_loop(0, n_ch, body, 0)

    return gather_kernel


def _proj_ln_body(g_ref, w_ref, gamma_ref, beta_ref, o_ref):
    g = g_ref[...]
    w = w_ref[...]
    h = jnp.dot(g, w, preferred_element_type=jnp.float32)
    mean = jnp.mean(h, axis=-1, keepdims=True)
    c = h - mean
    var = jnp.mean(c * c, axis=-1, keepdims=True)
    o_ref[...] = (c * lax.rsqrt(var + _EPS)) * gamma_ref[...] + beta_ref[...]


def _proj_ln(g, W, gamma, beta, T: int = 2048):
    N, D = g.shape
    H = W.shape[1]
    return pl.pallas_call(
        _proj_ln_body,
        grid=(N // T,),
        in_specs=[
            pl.BlockSpec((T, D), lambda i: (i, 0)),
            pl.BlockSpec((D, H), lambda i: (0, 0)),
            pl.BlockSpec((1, H), lambda i: (0, 0)),
            pl.BlockSpec((1, H), lambda i: (0, 0)),
        ],
        out_specs=pl.BlockSpec((T, H), lambda i: (i, 0)),
        out_shape=jax.ShapeDtypeStruct((N, H), jnp.float32),
    )(g, W, gamma.reshape(1, H), beta.reshape(1, H))


def kernel(entity_ids, table, W, gamma, beta):
    B, L = entity_ids.shape
    N = B * L
    V, D = table.shape
    H = W.shape[1]
    idx = entity_ids.T.reshape(N).astype(jnp.int32)
    g = _make_sc_gather(N, V, D)(idx, table)
    out = _proj_ln(g, W, gamma, beta)
    return out.reshape(L, B, H).transpose(1, 0, 2)


# speedup vs baseline: 6.0210x; 1.0505x over previous
import functools

import jax
import jax.numpy as jnp
from jax import lax
from jax.experimental import pallas as pl
from jax.experimental.pallas import tpu as pltpu
from jax.experimental.pallas import tpu_sc as plsc

_EPS = 1e-12


@functools.lru_cache(maxsize=None)
def _make_sc_gather(N: int, V: int, D: int):
    info = plsc.get_sparse_core_info()
    NC, NS = info.num_cores, info.num_subcores
    NW = NC * NS
    assert N % NW == 0
    b_per_w = N // NW
    CH = 640
    while b_per_w % CH:
        CH //= 2
    n_ch = b_per_w // CH
    mesh = plsc.VectorSubcoreMesh(core_axis_name="c", subcore_axis_name="s")

    @functools.partial(
        pl.kernel,
        mesh=mesh,
        out_type=jax.ShapeDtypeStruct((N, D), jnp.float32),
        scratch_types=[
            pltpu.VMEM((CH,), jnp.int32),
            pltpu.VMEM((CH, D), jnp.float32),
            pltpu.SemaphoreType.DMA,
        ],
    )
    def gather_kernel(idx_hbm, table_hbm, out_hbm, idx_v, rows_v, sem):
        wid = lax.axis_index("s") * NC + lax.axis_index("c")
        base = wid * b_per_w

        def body(i, carry):
            off = base + i * CH
            pltpu.sync_copy(idx_hbm.at[pl.ds(off, CH)], idx_v)
            pltpu.async_copy(table_hbm.at[idx_v], rows_v, sem).wait()
            pltpu.sync_copy(rows_v, out_hbm.at[pl.ds(off, CH)])
            return carry

        lax.fori_loop(0, n_ch, body, 0)

    return gather_kernel


def _proj_ln_body(g_ref, w_ref, gamma_ref, beta_ref, o_ref):
    g = g_ref[...]
    w = w_ref[...]
    h = jnp.dot(g, w, preferred_element_type=jnp.float32)
    mean = jnp.mean(h, axis=-1, keepdims=True)
    c = h - mean
    var = jnp.mean(c * c, axis=-1, keepdims=True)
    o_ref[...] = (c * lax.rsqrt(var + _EPS)) * gamma_ref[...] + beta_ref[...]


def _proj_ln(g, W, gamma, beta, T: int = 4096):
    N, D = g.shape
    H = W.shape[1]
    return pl.pallas_call(
        _proj_ln_body,
        grid=(N // T,),
        in_specs=[
            pl.BlockSpec((T, D), lambda i: (i, 0)),
            pl.BlockSpec((D, H), lambda i: (0, 0)),
            pl.BlockSpec((1, H), lambda i: (0, 0)),
            pl.BlockSpec((1, H), lambda i: (0, 0)),
        ],
        out_specs=pl.BlockSpec((T, H), lambda i: (i, 0)),
        out_shape=jax.ShapeDtypeStruct((N, H), jnp.float32),
    )(g, W, gamma.reshape(1, H), beta.reshape(1, H))


def kernel(entity_ids, table, W, gamma, beta):
    B, L = entity_ids.shape
    N = B * L
    V, D = table.shape
    H = W.shape[1]
    idx = entity_ids.T.reshape(N).astype(jnp.int32)
    g = _make_sc_gather(N, V, D)(idx, table)
    out = _proj_ln(g, W, gamma, beta)
    return out.reshape(L, B, H).transpose(1, 0, 2)
